# Initial kernel scaffold; baseline (speedup 1.0000x reference)
import jax
import jax.numpy as jnp
from jax.experimental import pallas as pl


def kernel(input, target):
    raise NotImplementedError("write your pallas kernel here")


# speedup vs baseline: 7.6193x; 7.6193x over previous
import jax
import jax.numpy as jnp
from jax import lax
from jax.experimental import pallas as pl
from jax.experimental.pallas import tpu as pltpu

_GAMMA = 2
_A0 = 0.25
_A1 = 0.75
_K = 39
_H = 512
_W = 512


def _shift_rows(a, d):
    z = jnp.zeros((abs(d), a.shape[1]), a.dtype)
    if d > 0:
        return jnp.concatenate([a[d:, :], z], axis=0)
    return jnp.concatenate([z, a[:d, :]], axis=0)


def _shift_cols(a, d):
    z = jnp.zeros((a.shape[0], abs(d)), a.dtype)
    if d > 0:
        return jnp.concatenate([a[:, d:], z], axis=1)
    return jnp.concatenate([z, a[:, :d]], axis=1)


def _body(x_ref, t_ref, out_ref, buf_ref, contrib_ref):
    b = pl.program_id(0)
    x = x_ref[0, 0]
    t = t_ref[0, 0]

    rs = t
    for d in (1, 2, -1, -2):
        rs = rs + _shift_rows(t, d)
    cs = rs
    for d in (1, 2, -1, -2):
        cs = cs + _shift_cols(rs, d)
    prot = cs > 0.0

    s = jnp.log1p(jnp.exp(-jnp.abs(x)))
    relu = jnp.maximum(x, 0.0)
    logpt = jnp.minimum(x, 0.0) - s
    logpt_bk = -relu - s
    pt = jnp.exp(logpt)
    pt_bk = 1.0 - jnp.exp(logpt_bk)

    focal_pos = -_A1 * (1.0 - pt) ** _GAMMA * logpt
    focal_neg = -_A0 * pt_bk ** _GAMMA * logpt_bk
    tpos = t > 0.0
    base = jnp.sum(jnp.where(tpos, focal_pos, 0.0))
    contrib_ref[...] = jnp.where(tpos, 0.0, focal_neg)

    bce = relu - x * t + s
    lp = jnp.where(prot, 0.0, bce)
    buf_ref[...] = lp
    row_max = jnp.max(lp, axis=1, keepdims=True)

    rio = lax.broadcasted_iota(jnp.int32, (_H, 1), 0)
    cio = lax.broadcasted_iota(jnp.int32, (1, _W), 1)

    def step(_, carry):
        row_max, acc = carry
        m = jnp.max(row_max)
        r = jnp.min(jnp.where(row_max == m, rio, _H))
        row = buf_ref[pl.ds(r, 1), :]
        c = jnp.min(jnp.where(row == m, cio, _W))
        crow = contrib_ref[pl.ds(r, 1), :]
        acc = acc + jnp.sum(jnp.where(cio == c, crow, 0.0))
        nrow = jnp.where(cio == c, -1.0, row)
        buf_ref[pl.ds(r, 1), :] = nrow
        row_max = jnp.where(rio == r, jnp.max(nrow), row_max)
        return row_max, acc

    _, acc = lax.fori_loop(0, _K, step, (row_max, jnp.float32(0.0)))

    @pl.when(b == 0)
    def _():
        out_ref[0, 0] = 0.0

    out_ref[0, 0] += base + acc


def kernel(input, target):
    res = pl.pallas_call(
        _body,
        grid=(input.shape[0],),
        in_specs=[
            pl.BlockSpec((1, 1, _H, _W), lambda b: (b, 0, 0, 0)),
            pl.BlockSpec((1, 1, _H, _W), lambda b: (b, 0, 0, 0)),
        ],
        out_specs=pl.BlockSpec((1, 1), lambda b: (0, 0),
                               memory_space=pltpu.SMEM),
        out_shape=jax.ShapeDtypeStruct((1, 1), jnp.float32),
        scratch_shapes=[
            pltpu.VMEM((_H, _W), jnp.float32),
            pltpu.VMEM((_H, _W), jnp.float32),
        ],
    )(input, target)
    return res[0, 0]


# speedup vs baseline: 17.0870x; 2.2426x over previous
import jax
import jax.numpy as jnp
from jax import lax
from jax.experimental import pallas as pl
from jax.experimental.pallas import tpu as pltpu
from jax.experimental.pallas import tpu_sc as plsc

_GAMMA = 2
_A0 = 0.25
_A1 = 0.75
_K = 39
_H = 512
_W = 512
_N = _H * _W
_B = 16

_CH = 4096
_NCHUNK = _N // _CH
_NG = _CH // 16
_NEG = -3.0e38
_POS = 3.0e38
_BIGI = 2**30


def _shift_rows(a, d):
    z = jnp.zeros((abs(d), a.shape[1]), a.dtype)
    if d > 0:
        return jnp.concatenate([a[d:, :], z], axis=0)
    return jnp.concatenate([z, a[:d, :]], axis=0)


def _shift_cols(a, d):
    z = jnp.zeros((a.shape[0], abs(d)), a.dtype)
    if d > 0:
        return jnp.concatenate([a[:, d:], z], axis=1)
    return jnp.concatenate([z, a[:, :d]], axis=1)


def _dense_body(x_ref, t_ref, base_ref, lp_ref):
    b = pl.program_id(0)
    x = x_ref[0, 0]
    t = t_ref[0, 0]

    rs = t
    for d in (1, 2, -1, -2):
        rs = rs + _shift_rows(t, d)
    cs = rs
    for d in (1, 2, -1, -2):
        cs = cs + _shift_cols(rs, d)
    prot = cs > 0.0

    s = jnp.log1p(jnp.exp(-jnp.abs(x)))
    relu = jnp.maximum(x, 0.0)
    logpt = jnp.minimum(x, 0.0) - s
    pt = jnp.exp(logpt)
    focal_pos = -_A1 * (1.0 - pt) ** _GAMMA * logpt
    base = jnp.sum(jnp.where(t > 0.0, focal_pos, 0.0))

    bce = relu - x * t + s
    lp_ref[0] = jnp.where(prot, 0.0, bce)

    @pl.when(b == 0)
    def _():
        base_ref[0, 0] = 0.0

    base_ref[0, 0] += base


def _dense(input, target):
    return pl.pallas_call(
        _dense_body,
        grid=(_B,),
        in_specs=[
            pl.BlockSpec((1, 1, _H, _W), lambda b: (b, 0, 0, 0)),
            pl.BlockSpec((1, 1, _H, _W), lambda b: (b, 0, 0, 0)),
        ],
        out_specs=[
            pl.BlockSpec((1, 1), lambda b: (0, 0), memory_space=pltpu.SMEM),
            pl.BlockSpec((1, _H, _W), lambda b: (b, 0, 0)),
        ],
        out_shape=[
            jax.ShapeDtypeStruct((1, 1), jnp.float32),
            jax.ShapeDtypeStruct((_B, _H, _W), jnp.float32),
        ],
    )(input, target)


def _lane_shuffle_reduce(v, op):
    i0 = lax.broadcasted_iota(jnp.int32, (16,), 0)
    for sh in (8, 4, 2, 1):
        perm = jnp.bitwise_xor(i0, sh)
        v = op(v, v.at[perm].get(mode="promise_in_bounds"))
    return v


def _rmax(v):
    return _lane_shuffle_reduce(v, jnp.maximum)[0]


def _rmin(v):
    return _lane_shuffle_reduce(v, jnp.minimum)[0]


def _rsum(v):
    return _lane_shuffle_reduce(v, jnp.add)[0]


def _sc_body(loss_hbm, x2_hbm, t2_hbm, out_hbm,
             buf0, buf1, vals_v, idxs_v, rowi_v, xrows_v, trows_v,
             orow_v, sem0, sem1, thr_s, evi_s):
    c = lax.axis_index("c")
    s = lax.axis_index("s")
    img = c * 8 + s
    lio = lax.broadcasted_iota(jnp.int32, (16,), 0)

    def dma(off, buf, sem):
        return pltpu.make_async_copy(
            loss_hbm.at[img, pl.ds(off, _CH)], buf, sem)

    def insert(v, i):
        hit = (v > thr_s[0]) | ((v == thr_s[0]) & (i < evi_s[0]))

        @pl.when(hit)
        def _():
            va = [vals_v[pl.ds(16 * k, 16)] for k in range(3)]
            ia = [idxs_v[pl.ds(16 * k, 16)] for k in range(3)]
            mm = _rmin(jnp.minimum(jnp.minimum(va[0], va[1]), va[2]))
            sel = _rmax(jnp.maximum(
                jnp.maximum(jnp.where(va[0] == mm, ia[0], -1),
                            jnp.where(va[1] == mm, ia[1], -1)),
                jnp.where(va[2] == mm, ia[2], -1)))
            nv, ni = [], []
            for k in range(3):
                mk = (va[k] == mm) & (ia[k] == sel)
                nv.append(jnp.where(mk, v, va[k]))
                ni.append(jnp.where(mk, i, ia[k]))
                vals_v[pl.ds(16 * k, 16)] = nv[k]
                idxs_v[pl.ds(16 * k, 16)] = ni[k]
            mm2 = _rmin(jnp.minimum(jnp.minimum(nv[0], nv[1]), nv[2]))
            sel2 = _rmax(jnp.maximum(
                jnp.maximum(jnp.where(nv[0] == mm2, ni[0], -1),
                            jnp.where(nv[1] == mm2, ni[1], -1)),
                jnp.where(nv[2] == mm2, ni[2], -1)))
            thr_s[0] = mm2
            evi_s[0] = sel2

    def process(buf, off):
        def g16(i, vm):
            gb = i * 256
            for j in range(16):
                vm = jnp.maximum(vm, buf[pl.ds(gb + j * 16, 16)])
            return vm
        vm = lax.fori_loop(0, _NG // 16, g16,
                           jnp.full((16,), _NEG, jnp.float32))
        cmax = _rmax(vm)
        chit = (cmax > thr_s[0]) | ((cmax == thr_s[0]) & (off <= evi_s[0]))

        @pl.when(chit)
        def _():
            def grp(gi, _):
                gb = gi * 16
                v = buf[pl.ds(gb, 16)]
                gm = _rmax(v)
                ghit = ((gm > thr_s[0])
                        | ((gm == thr_s[0]) & (off + gb <= evi_s[0])))

                @pl.when(ghit)
                def _():
                    for j in range(16):
                        insert(v[j], off + gb + j)
                return 0
            lax.fori_loop(0, _NG, grp, 0)

    @pl.when(s < 8)
    def _():
        for k in range(3):
            active = (lio + 16 * k) < _K
            vals_v[pl.ds(16 * k, 16)] = jnp.where(active, _NEG, _POS)
            idxs_v[pl.ds(16 * k, 16)] = jnp.where(active, _BIGI, 0)
        thr_s[0] = _NEG
        evi_s[0] = _BIGI

        dma(0, buf0, sem0).start()
        dma(_CH, buf1, sem1).start()

        def pair(m, _):
            o0 = (2 * m) * _CH
            dma(o0, buf0, sem0).wait()
            process(buf0, o0)

            @pl.when(2 * m + 2 < _NCHUNK)
            def _():
                dma(o0 + 2 * _CH, buf0, sem0).start()

            o1 = o0 + _CH
            dma(o1, buf1, sem1).wait()
            process(buf1, o1)

            @pl.when(2 * m + 3 < _NCHUNK)
            def _():
                dma(o1 + 2 * _CH, buf1, sem1).start()
            return 0

        lax.fori_loop(0, _NCHUNK // 2, pair, 0)

        gbase = img * _N
        for k in range(3):
            idx = idxs_v[pl.ds(16 * k, 16)]
            rowi_v[pl.ds(16 * k, 16)] = gbase + idx
        pltpu.async_copy(x2_hbm.at[rowi_v], xrows_v, sem0).wait()
        pltpu.async_copy(t2_hbm.at[rowi_v], trows_v, sem1).wait()

        total = jnp.float32(0.0)
        for k in range(3):
            rloc = lio + 16 * k
            xv = xrows_v[pl.ds(16 * k, 16)]
            tv = trows_v[pl.ds(16 * k, 16)]
            u = jnp.exp(-jnp.abs(xv))
            z = u / (2.0 + u)
            z2 = z * z
            log1pu = 2.0 * z * (1.0 + z2 * (
                jnp.float32(1 / 3) + z2 * (jnp.float32(1 / 5) + z2 * (
                    jnp.float32(1 / 7) + z2 * jnp.float32(1 / 9)))))
            logpt_bk = -jnp.maximum(xv, 0.0) - log1pu
            pt_bk = 1.0 - jnp.exp(logpt_bk)
            fneg = _A0 * pt_bk * pt_bk * (-logpt_bk)
            contrib = jnp.where((tv > 0.0) | (rloc >= _K), 0.0, fneg)
            total = total + _rsum(contrib)

        orow_v[...] = jnp.where(lio == 0, total, 0.0)
        pltpu.sync_copy(orow_v, out_hbm.at[img])


def _sc_topk(loss2, x2, t2):
    mesh = plsc.VectorSubcoreMesh(core_axis_name="c", subcore_axis_name="s")
    return pl.kernel(
        _sc_body,
        out_type=jax.ShapeDtypeStruct((_B, 16), jnp.float32),
        mesh=mesh,
        scratch_types=[
            pltpu.VMEM((_CH,), jnp.float32),
            pltpu.VMEM((_CH,), jnp.float32),
            pltpu.VMEM((48,), jnp.float32),
            pltpu.VMEM((48,), jnp.int32),
            pltpu.VMEM((48,), jnp.int32),
            pltpu.VMEM((48,), jnp.float32),
            pltpu.VMEM((48,), jnp.float32),
            pltpu.VMEM((16,), jnp.float32),
            pltpu.SemaphoreType.DMA,
            pltpu.SemaphoreType.DMA,
            pltpu.SMEM((1,), jnp.float32),
            pltpu.SMEM((1,), jnp.int32),
        ],
    )(loss2, x2, t2)


def kernel(input, target):
    base, lp = _dense(input, target)
    loss2 = lp.reshape(_B, _N)
    x2 = input.reshape(_B * _N)
    t2 = target.reshape(_B * _N)
    part = _sc_topk(loss2, x2, t2)
    return base[0, 0] + jnp.sum(part[:, 0])


# speedup vs baseline: 24.6985x; 1.4455x over previous
import jax
import jax.numpy as jnp
from jax import lax
from jax.experimental import pallas as pl
from jax.experimental.pallas import tpu as pltpu
from jax.experimental.pallas import tpu_sc as plsc

_GAMMA = 2
_A0 = 0.25
_A1 = 0.75
_K = 39
_H = 512
_W = 512
_N = _H * _W
_B = 16

_CH = 4096
_NCHUNK = _N // _CH
_NG = _CH // 16
_NEG = -3.0e38
_POS = 3.0e38
_BIGI = 2**30


def _shift_rows(a, d):
    z = jnp.zeros((abs(d), a.shape[1]), a.dtype)
    if d > 0:
        return jnp.concatenate([a[d:, :], z], axis=0)
    return jnp.concatenate([z, a[:d, :]], axis=0)


def _shift_cols(a, d):
    z = jnp.zeros((a.shape[0], abs(d)), a.dtype)
    if d > 0:
        return jnp.concatenate([a[:, d:], z], axis=1)
    return jnp.concatenate([z, a[:, :d]], axis=1)


def _dense_body(x_ref, t_ref, base_ref, lp_ref, ct_ref):
    b = pl.program_id(0)
    x = x_ref[0, 0]
    t = t_ref[0, 0]

    rs = t
    for d in (1, 2, -1, -2):
        rs = rs + _shift_rows(t, d)
    cs = rs
    for d in (1, 2, -1, -2):
        cs = cs + _shift_cols(rs, d)
    prot = cs > 0.0

    s = jnp.log1p(jnp.exp(-jnp.abs(x)))
    relu = jnp.maximum(x, 0.0)
    logpt = jnp.minimum(x, 0.0) - s
    logpt_bk = -relu - s
    pt = jnp.exp(logpt)
    pt_bk = 1.0 - jnp.exp(logpt_bk)
    focal_pos = -_A1 * (1.0 - pt) ** _GAMMA * logpt
    focal_neg = -_A0 * pt_bk ** _GAMMA * logpt_bk
    base = jnp.sum(jnp.where(t > 0.0, focal_pos, 0.0))

    bce = relu - x * t + s
    lp_ref[...] = jnp.reshape(jnp.where(prot, 0.0, bce), (_N,))
    ct_ref[...] = jnp.reshape(jnp.where(t > 0.0, 0.0, focal_neg), (_N,))

    @pl.when(b == 0)
    def _():
        base_ref[0, 0] = 0.0

    base_ref[0, 0] += base


def _dense(input, target):
    return pl.pallas_call(
        _dense_body,
        grid=(_B,),
        in_specs=[
            pl.BlockSpec((1, 1, _H, _W), lambda b: (b, 0, 0, 0)),
            pl.BlockSpec((1, 1, _H, _W), lambda b: (b, 0, 0, 0)),
        ],
        out_specs=[
            pl.BlockSpec((1, 1), lambda b: (0, 0), memory_space=pltpu.SMEM),
            pl.BlockSpec((_N,), lambda b: (b,)),
            pl.BlockSpec((_N,), lambda b: (b,)),
        ],
        out_shape=[
            jax.ShapeDtypeStruct((1, 1), jnp.float32),
            jax.ShapeDtypeStruct((_B * _N,), jnp.float32),
            jax.ShapeDtypeStruct((_B * _N,), jnp.float32),
        ],
    )(input, target)


def _lane_shuffle_reduce(v, op):
    i0 = lax.broadcasted_iota(jnp.int32, (16,), 0)
    for sh in (8, 4, 2, 1):
        perm = jnp.bitwise_xor(i0, sh)
        v = op(v, v.at[perm].get(mode="promise_in_bounds"))
    return v


def _rmax(v):
    return _lane_shuffle_reduce(v, jnp.maximum)[0]


def _rmin(v):
    return _lane_shuffle_reduce(v, jnp.minimum)[0]


def _rsum(v):
    return _lane_shuffle_reduce(v, jnp.add)[0]


def _sc_body(loss_hbm, ct_hbm, out_hbm,
             buf0, buf1, vals_v, idxs_v, rowi_v, cg_v,
             orow_v, sem0, sem1, thr_s, evi_s):
    c = lax.axis_index("c")
    s = lax.axis_index("s")
    img = c * 8 + s
    lio = lax.broadcasted_iota(jnp.int32, (16,), 0)

    gbase = img * _N

    def dma(off, buf, sem):
        return pltpu.make_async_copy(
            loss_hbm.at[pl.ds(gbase + off, _CH)], buf, sem)

    def insert(v, i):
        hit = (v > thr_s[0]) | ((v == thr_s[0]) & (i < evi_s[0]))

        @pl.when(hit)
        def _():
            va = [vals_v[pl.ds(16 * k, 16)] for k in range(3)]
            ia = [idxs_v[pl.ds(16 * k, 16)] for k in range(3)]
            mm = _rmin(jnp.minimum(jnp.minimum(va[0], va[1]), va[2]))
            sel = _rmax(jnp.maximum(
                jnp.maximum(jnp.where(va[0] == mm, ia[0], -1),
                            jnp.where(va[1] == mm, ia[1], -1)),
                jnp.where(va[2] == mm, ia[2], -1)))
            nv, ni = [], []
            for k in range(3):
                mk = (va[k] == mm) & (ia[k] == sel)
                nv.append(jnp.where(mk, v, va[k]))
                ni.append(jnp.where(mk, i, ia[k]))
                vals_v[pl.ds(16 * k, 16)] = nv[k]
                idxs_v[pl.ds(16 * k, 16)] = ni[k]
            mm2 = _rmin(jnp.minimum(jnp.minimum(nv[0], nv[1]), nv[2]))
            sel2 = _rmax(jnp.maximum(
                jnp.maximum(jnp.where(nv[0] == mm2, ni[0], -1),
                            jnp.where(nv[1] == mm2, ni[1], -1)),
                jnp.where(nv[2] == mm2, ni[2], -1)))
            thr_s[0] = mm2
            evi_s[0] = sel2

    def process(buf, off):
        def g16(i, vm):
            gb = i * 256
            for j in range(16):
                vm = jnp.maximum(vm, buf[pl.ds(gb + j * 16, 16)])
            return vm
        vm = lax.fori_loop(0, _NG // 16, g16,
                           jnp.full((16,), _NEG, jnp.float32))
        cmax = _rmax(vm)
        chit = (cmax > thr_s[0]) | ((cmax == thr_s[0]) & (off <= evi_s[0]))

        @pl.when(chit)
        def _():
            def grp(gi, _):
                gb = gi * 16
                v = buf[pl.ds(gb, 16)]
                gm = _rmax(v)
                ghit = ((gm > thr_s[0])
                        | ((gm == thr_s[0]) & (off + gb <= evi_s[0])))

                @pl.when(ghit)
                def _():
                    for j in range(16):
                        insert(v[j], off + gb + j)
                return 0
            lax.fori_loop(0, _NG, grp, 0)

    @pl.when(s < 8)
    def _():
        for k in range(3):
            active = (lio + 16 * k) < _K
            vals_v[pl.ds(16 * k, 16)] = jnp.where(active, _NEG, _POS)
            idxs_v[pl.ds(16 * k, 16)] = jnp.where(active, _BIGI, 0)
        thr_s[0] = _NEG
        evi_s[0] = _BIGI

        dma(0, buf0, sem0).start()
        dma(_CH, buf1, sem1).start()

        def pair(m, _):
            o0 = (2 * m) * _CH
            dma(o0, buf0, sem0).wait()
            process(buf0, o0)

            @pl.when(2 * m + 2 < _NCHUNK)
            def _():
                dma(o0 + 2 * _CH, buf0, sem0).start()

            o1 = o0 + _CH
            dma(o1, buf1, sem1).wait()
            process(buf1, o1)

            @pl.when(2 * m + 3 < _NCHUNK)
            def _():
                dma(o1 + 2 * _CH, buf1, sem1).start()
            return 0

        lax.fori_loop(0, _NCHUNK // 2, pair, 0)

        for k in range(3):
            idx = idxs_v[pl.ds(16 * k, 16)]
            rowi_v[pl.ds(16 * k, 16)] = gbase + idx
        pltpu.async_copy(ct_hbm.at[rowi_v], cg_v, sem0).wait()

        total = jnp.float32(0.0)
        for k in range(3):
            rloc = lio + 16 * k
            cv = cg_v[pl.ds(16 * k, 16)]
            contrib = jnp.where(rloc >= _K, 0.0, cv)
            total = total + _rsum(contrib)

        orow_v[...] = jnp.where(lio == 0, total, 0.0)
        pltpu.sync_copy(orow_v, out_hbm.at[img])


def _sc_topk(loss1, ct1):
    mesh = plsc.VectorSubcoreMesh(core_axis_name="c", subcore_axis_name="s")
    return pl.kernel(
        _sc_body,
        out_type=jax.ShapeDtypeStruct((_B, 16), jnp.float32),
        mesh=mesh,
        scratch_types=[
            pltpu.VMEM((_CH,), jnp.float32),
            pltpu.VMEM((_CH,), jnp.float32),
            pltpu.VMEM((48,), jnp.float32),
            pltpu.VMEM((48,), jnp.int32),
            pltpu.VMEM((48,), jnp.int32),
            pltpu.VMEM((48,), jnp.float32),
            pltpu.VMEM((16,), jnp.float32),
            pltpu.SemaphoreType.DMA,
            pltpu.SemaphoreType.DMA,
            pltpu.SMEM((1,), jnp.float32),
            pltpu.SMEM((1,), jnp.int32),
        ],
    )(loss1, ct1)


def kernel(input, target):
    base, lp1, ct1 = _dense(input, target)
    part = _sc_topk(lp1, ct1)
    return base[0, 0] + jnp.sum(part[:, 0])


# speedup vs baseline: 27.1749x; 1.1003x over previous
import jax
import jax.numpy as jnp
from jax import lax
from jax.experimental import pallas as pl
from jax.experimental.pallas import tpu as pltpu
from jax.experimental.pallas import tpu_sc as plsc

_GAMMA = 2
_A0 = 0.25
_A1 = 0.75
_K = 39
_H = 512
_W = 512
_N = _H * _W
_B = 16

_CH = 4096
_NCHUNK = _N // _CH
_NG = _CH // 16
_NEG = -3.0e38
_POS = 3.0e38
_BIGI = 2**30


def _shift_rows(a, d):
    z = jnp.zeros((abs(d), a.shape[1]), a.dtype)
    if d > 0:
        return jnp.concatenate([a[d:, :], z], axis=0)
    return jnp.concatenate([z, a[:d, :]], axis=0)


def _shift_cols(a, d):
    z = jnp.zeros((a.shape[0], abs(d)), a.dtype)
    if d > 0:
        return jnp.concatenate([a[:, d:], z], axis=1)
    return jnp.concatenate([z, a[:, :d]], axis=1)


def _dense_body(x_ref, t_ref, base_ref, lp_ref, ct_ref):
    b = pl.program_id(0)
    x = x_ref[0, 0]
    t = t_ref[0, 0]

    rs = t
    for d in (1, 2, -1, -2):
        rs = rs + _shift_rows(t, d)
    cs = rs
    for d in (1, 2, -1, -2):
        cs = cs + _shift_cols(rs, d)
    prot = cs > 0.0

    s = jnp.log1p(jnp.exp(-jnp.abs(x)))
    relu = jnp.maximum(x, 0.0)
    logpt = jnp.minimum(x, 0.0) - s
    logpt_bk = -relu - s
    pt = jnp.exp(logpt)
    pt_bk = 1.0 - jnp.exp(logpt_bk)
    focal_pos = -_A1 * (1.0 - pt) ** _GAMMA * logpt
    focal_neg = -_A0 * pt_bk ** _GAMMA * logpt_bk
    base = jnp.sum(jnp.where(t > 0.0, focal_pos, 0.0))

    bce = relu - x * t + s
    lp_ref[...] = jnp.reshape(jnp.where(prot, 0.0, bce), (_N,))
    ct_ref[...] = jnp.reshape(jnp.where(t > 0.0, 0.0, focal_neg), (_N,))

    @pl.when(b == 0)
    def _():
        base_ref[0, 0] = 0.0

    base_ref[0, 0] += base


def _dense(input, target):
    return pl.pallas_call(
        _dense_body,
        grid=(_B,),
        in_specs=[
            pl.BlockSpec((1, 1, _H, _W), lambda b: (b, 0, 0, 0)),
            pl.BlockSpec((1, 1, _H, _W), lambda b: (b, 0, 0, 0)),
        ],
        out_specs=[
            pl.BlockSpec((1, 1), lambda b: (0, 0), memory_space=pltpu.SMEM),
            pl.BlockSpec((_N,), lambda b: (b,)),
            pl.BlockSpec((_N,), lambda b: (b,)),
        ],
        out_shape=[
            jax.ShapeDtypeStruct((1, 1), jnp.float32),
            jax.ShapeDtypeStruct((_B * _N,), jnp.float32),
            jax.ShapeDtypeStruct((_B * _N,), jnp.float32),
        ],
    )(input, target)


def _lane_shuffle_reduce(v, op):
    i0 = lax.broadcasted_iota(jnp.int32, (16,), 0)
    for sh in (8, 4, 2, 1):
        perm = jnp.bitwise_xor(i0, sh)
        v = op(v, v.at[perm].get(mode="promise_in_bounds"))
    return v


def _rmax(v):
    return _lane_shuffle_reduce(v, jnp.maximum)[0]


def _rmin(v):
    return _lane_shuffle_reduce(v, jnp.minimum)[0]


def _rsum(v):
    return _lane_shuffle_reduce(v, jnp.add)[0]


def _sc_body(loss_hbm, ct_hbm, out_hbm,
             buf0, buf1, vals_v, idxs_v, rowi_v, cg_v,
             orow_v, pv_v, pi_v, shv_sp, shi_sp, sem0, sem1, thr_s, evi_s):
    c = lax.axis_index("c")
    s = lax.axis_index("s")
    img = c * 8 + lax.shift_right_logical(s, 1)
    half = jnp.bitwise_and(s, 1)
    lio = lax.broadcasted_iota(jnp.int32, (16,), 0)

    gbase = img * _N
    hoff = half * (_N // 2)

    def dma(off, buf, sem):
        return pltpu.make_async_copy(
            loss_hbm.at[pl.ds(gbase + off, _CH)], buf, sem)

    def insert(v, i):
        hit = (v > thr_s[0]) | ((v == thr_s[0]) & (i < evi_s[0]))

        @pl.when(hit)
        def _():
            va = [vals_v[pl.ds(16 * k, 16)] for k in range(3)]
            ia = [idxs_v[pl.ds(16 * k, 16)] for k in range(3)]
            mm = _rmin(jnp.minimum(jnp.minimum(va[0], va[1]), va[2]))
            sel = _rmax(jnp.maximum(
                jnp.maximum(jnp.where(va[0] == mm, ia[0], -1),
                            jnp.where(va[1] == mm, ia[1], -1)),
                jnp.where(va[2] == mm, ia[2], -1)))
            nv, ni = [], []
            for k in range(3):
                mk = (va[k] == mm) & (ia[k] == sel)
                nv.append(jnp.where(mk, v, va[k]))
                ni.append(jnp.where(mk, i, ia[k]))
                vals_v[pl.ds(16 * k, 16)] = nv[k]
                idxs_v[pl.ds(16 * k, 16)] = ni[k]
            mm2 = _rmin(jnp.minimum(jnp.minimum(nv[0], nv[1]), nv[2]))
            sel2 = _rmax(jnp.maximum(
                jnp.maximum(jnp.where(nv[0] == mm2, ni[0], -1),
                            jnp.where(nv[1] == mm2, ni[1], -1)),
                jnp.where(nv[2] == mm2, ni[2], -1)))
            thr_s[0] = mm2
            evi_s[0] = sel2

    def process(buf, off):
        def g16(i, vm):
            gb = i * 256
            for j in range(16):
                vm = jnp.maximum(vm, buf[pl.ds(gb + j * 16, 16)])
            return vm
        vm = lax.fori_loop(0, _NG // 16, g16,
                           jnp.full((16,), _NEG, jnp.float32))
        cmax = _rmax(vm)
        chit = (cmax > thr_s[0]) | ((cmax == thr_s[0]) & (off <= evi_s[0]))

        @pl.when(chit)
        def _():
            def grp(gi, _):
                gb = gi * 16
                v = buf[pl.ds(gb, 16)]
                gm = _rmax(v)
                ghit = ((gm > thr_s[0])
                        | ((gm == thr_s[0]) & (off + gb <= evi_s[0])))

                @pl.when(ghit)
                def _():
                    for j in range(16):
                        insert(v[j], off + gb + j)
                return 0
            lax.fori_loop(0, _NG, grp, 0)

    nch = _NCHUNK // 2

    for k in range(3):
        active = (lio + 16 * k) < _K
        vals_v[pl.ds(16 * k, 16)] = jnp.where(active, _NEG, _POS)
        idxs_v[pl.ds(16 * k, 16)] = jnp.where(active, _BIGI, 0)
    thr_s[0] = _NEG
    evi_s[0] = _BIGI

    dma(hoff, buf0, sem0).start()
    dma(hoff + _CH, buf1, sem1).start()

    def pair(m, _):
        o0 = hoff + (2 * m) * _CH
        dma(o0, buf0, sem0).wait()
        process(buf0, o0)

        @pl.when(2 * m + 2 < nch)
        def _():
            dma(o0 + 2 * _CH, buf0, sem0).start()

        o1 = o0 + _CH
        dma(o1, buf1, sem1).wait()
        process(buf1, o1)

        @pl.when(2 * m + 3 < nch)
        def _():
            dma(o1 + 2 * _CH, buf1, sem1).start()
        return 0

    lax.fori_loop(0, nch // 2, pair, 0)

    pltpu.sync_copy(vals_v, shv_sp.at[s])
    pltpu.sync_copy(idxs_v, shi_sp.at[s])
    plsc.subcore_barrier()

    @pl.when(half == 0)
    def _():
        pltpu.sync_copy(shv_sp.at[s + 1], pv_v)
        pltpu.sync_copy(shi_sp.at[s + 1], pi_v)
        for k in range(3):
            if 16 * k >= _K:
                break
            pvk = pv_v[pl.ds(16 * k, 16)]
            pik = pi_v[pl.ds(16 * k, 16)]
            for j in range(16):
                if 16 * k + j < _K:
                    insert(pvk[j], pik[j])

        for k in range(3):
            idx = idxs_v[pl.ds(16 * k, 16)]
            rowi_v[pl.ds(16 * k, 16)] = gbase + idx
        pltpu.async_copy(ct_hbm.at[rowi_v], cg_v, sem0).wait()

        total = jnp.float32(0.0)
        for k in range(3):
            rloc = lio + 16 * k
            cv = cg_v[pl.ds(16 * k, 16)]
            contrib = jnp.where(rloc >= _K, 0.0, cv)
            total = total + _rsum(contrib)

        orow_v[...] = jnp.where(lio == 0, total, 0.0)
        pltpu.sync_copy(orow_v, out_hbm.at[img])


def _sc_topk(loss1, ct1):
    mesh = plsc.VectorSubcoreMesh(core_axis_name="c", subcore_axis_name="s")
    return pl.kernel(
        _sc_body,
        out_type=jax.ShapeDtypeStruct((_B, 16), jnp.float32),
        mesh=mesh,
        scratch_types=[
            pltpu.VMEM((_CH,), jnp.float32),
            pltpu.VMEM((_CH,), jnp.float32),
            pltpu.VMEM((48,), jnp.float32),
            pltpu.VMEM((48,), jnp.int32),
            pltpu.VMEM((48,), jnp.int32),
            pltpu.VMEM((48,), jnp.float32),
            pltpu.VMEM((16,), jnp.float32),
            pltpu.VMEM((48,), jnp.float32),
            pltpu.VMEM((48,), jnp.int32),
            pltpu.VMEM_SHARED((16, 48), jnp.float32),
            pltpu.VMEM_SHARED((16, 48), jnp.int32),
            pltpu.SemaphoreType.DMA,
            pltpu.SemaphoreType.DMA,
            pltpu.SMEM((1,), jnp.float32),
            pltpu.SMEM((1,), jnp.int32),
        ],
    )(loss1, ct1)


def kernel(input, target):
    base, lp1, ct1 = _dense(input, target)
    part = _sc_topk(lp1, ct1)
    return base[0, 0] + jnp.sum(part[:, 0])
